# baseline (speedup 1.0000x reference)
import functools

import jax
import jax.numpy as jnp
from jax.experimental import pallas as pl

EPS_DICE = 0.0001


def _dice_block_kernel(out_ref, tgt_ref, loss_ref):
    x = out_ref[0]
    t = tgt_ref[0]
    c = x.shape[0]
    s = jax.nn.sigmoid(x)
    sig_sum = jnp.sum(s, axis=(1, 2))
    cls = jax.lax.broadcasted_iota(jnp.int32, x.shape, 0)
    mask = t[None, :, :] == cls
    inter = jnp.sum(jnp.where(mask, s, 0.0), axis=(1, 2))
    cnt = jnp.sum(mask.astype(jnp.float32), axis=(1, 2))
    loss_ref[0, 0] = 1.0 - 2.0 * inter / (sig_sum + cnt + EPS_DICE)


@jax.jit
def kernel(output, target):
    b, c, h, w = output.shape
    tgt = target.astype(jnp.int32)
    loss_per_channel = pl.pallas_call(
        _dice_block_kernel,
        grid=(b,),
        in_specs=[
            pl.BlockSpec((1, c, h, w), lambda i: (i, 0, 0, 0)),
            pl.BlockSpec((1, h, w), lambda i: (i, 0, 0)),
        ],
        out_specs=pl.BlockSpec((1, 1, c), lambda i: (i, 0, 0)),
        out_shape=jax.ShapeDtypeStruct((b, 1, c), jnp.float32),
    )(output, tgt)
    return loss_per_channel.sum(axis=(1, 2)) / c
